# baseline (speedup 1.0000x reference)
import functools

import jax
import jax.numpy as jnp
from jax.experimental import pallas as pl
from jax.experimental.pallas import tpu as pltpu
from jax.experimental.pallas import tpu_sc as plsc

H = 8
N = 256
C = 64
ROWS = 8 * N
K = 32

_NW = 32
_SLICES = (640, 640, 640, 128)
_GRP = N // 16


def _stage1_body(q_ref, k_ref, sq_ref, sk_ref, roi_ref, w_ref, b_ref,
                 out1_ref, sc_ref):
    q = q_ref[...]
    k = k_ref[...]
    rb = q.shape[0]
    dot = jnp.sum(q * k, axis=1)
    qn = jnp.maximum(jnp.sqrt(jnp.sum(q * q, axis=1)), 1e-8)
    kn = jnp.maximum(jnp.sqrt(jnp.sum(k * k, axis=1)), 1e-8)
    scores = jnp.maximum(dot / (qn * kn), 0.0)
    sc_ref[...] = scores.reshape(2 * rb, N // 2)

    w = w_ref[...]
    wq = jnp.broadcast_to(w[None, :, 0:C], (rb, H, C))
    wk = jnp.broadcast_to(w[None, :, C:2 * C], (rb, H, C))
    z = jax.lax.dot_general(wq, q, (((2,), (1,)), ((0,), (0,))),
                            preferred_element_type=jnp.float32)
    z = z + jax.lax.dot_general(wk, k, (((2,), (1,)), ((0,), (0,))),
                                preferred_element_type=jnp.float32)
    sq = sq_ref[...]
    sk = sk_ref[...]
    z = (z
         + w[None, :, 2 * C:2 * C + 1] * sq[:, 0:1, :]
         + w[None, :, 2 * C + 1:2 * C + 2] * sq[:, 1:2, :]
         + w[None, :, 2 * C + 2:2 * C + 3] * sk[:, 0:1, :]
         + w[None, :, 2 * C + 3:2 * C + 4] * sk[:, 1:2, :])
    z = z + b_ref[...][None, :, :]
    attn1 = 1.0 / (1.0 + jnp.exp(-z))
    out1_ref[...] = (attn1 * roi_ref[...][:, None, :]).astype(jnp.bfloat16)


def _dyn_gather(x, idx):
    return jax.lax.gather(
        x, idx[:, None],
        dimension_numbers=jax.lax.GatherDimensionNumbers(
            offset_dims=(), collapsed_slice_dims=(0,), start_index_map=(0,)),
        slice_sizes=(1,), mode=jax.lax.GatherScatterMode.PROMISE_IN_BOUNDS)


def _vsum16(v):
    i = jax.lax.iota(jnp.int32, 16)
    for s in (8, 4, 2, 1):
        v = v + _dyn_gather(v, i ^ s)
    return v


def _prefix16(v):
    i = jax.lax.iota(jnp.int32, 16)
    for s in (1, 2, 4, 8):
        sh = _dyn_gather(v, jnp.maximum(i - s, 0))
        v = v + jnp.where(i >= s, sh, 0)
    return v


def _sc_topk_union_body(rpw, scores_hbm, out_hbm, rows_v, buf_v):
    wid = jax.lax.axis_index("c") * 16 + jax.lax.axis_index("s")
    pltpu.sync_copy(scores_hbm.at[pl.ds(wid * (rpw * N), rpw * N)], rows_v)
    lane15 = jnp.full((16,), 15, jnp.int32)

    def row_body(r, acc):
        start = r * N
        bs = [jax.lax.bitcast_convert_type(rows_v[pl.ds(start + g * 16, 16)],
                                           jnp.int32)
              for g in range(_GRP)]

        def bit_body(i, t):
            cand = t | jax.lax.shift_left(jnp.int32(1), jnp.int32(30) - i)
            cnt = jnp.zeros((16,), jnp.int32)
            for bvec in bs:
                cnt = cnt + jnp.where(bvec >= cand, 1, 0)
            return jnp.where(_vsum16(cnt) >= K, cand, t)

        t = jax.lax.fori_loop(0, 31, bit_body, jnp.zeros((16,), jnp.int32))

        cg = jnp.zeros((16,), jnp.int32)
        for bvec in bs:
            cg = cg + jnp.where(bvec > t, 1, 0)
        rem = K - _vsum16(cg)
        new_acc = []
        c = jnp.zeros((16,), jnp.int32)
        for g in range(_GRP):
            eq = bs[g] == t
            pref = _prefix16(jnp.where(eq, 1, 0)) + c
            keep = (bs[g] > t) | (eq & (pref <= rem))
            new_acc.append(jnp.maximum(acc[g], jnp.where(keep, 1.0, 0.0)))
            c = _dyn_gather(pref, lane15)
        return tuple(new_acc)

    acc0 = tuple(jnp.zeros((16,), jnp.float32) for _ in range(_GRP))
    acc = jax.lax.fori_loop(0, rpw, row_body, acc0)
    for g in range(_GRP):
        buf_v[pl.ds(g * 16, 16)] = acc[g]
    pltpu.sync_copy(buf_v, out_hbm.at[pl.ds(wid * N, N)])


@functools.cache
def _sc_topk_union(rpw):
    return pl.kernel(
        functools.partial(_sc_topk_union_body, rpw),
        out_type=jax.ShapeDtypeStruct((_NW * N,), jnp.float32),
        mesh=plsc.VectorSubcoreMesh(core_axis_name="c", subcore_axis_name="s"),
        scratch_types=[
            pltpu.VMEM((rpw * N,), jnp.float32),
            pltpu.VMEM((N,), jnp.float32),
        ],
    )


def _stage3_body(out1_ref, m_ref, o_ref):
    colmask = jnp.max(m_ref[...], axis=0)
    o_ref[...] = out1_ref[...].astype(jnp.float32) * colmask[None, None, :]


def _stage3_acc_body(prev_ref, out1_ref, m_ref, o_ref):
    del prev_ref
    _stage3_body(out1_ref, m_ref, o_ref)


def kernel(query, key_t, s_query, s_key, roi_mask, W, b, node_num):
    B = query.shape[0]
    q = query.transpose(0, 1, 3, 2).reshape(ROWS, C, N)
    k = key_t.transpose(0, 1, 3, 2).reshape(ROWS, C, N)
    sq = s_query.transpose(0, 1, 3, 2).reshape(ROWS, 2, N)
    sk = s_key.transpose(0, 1, 3, 2).reshape(ROWS, 2, N)
    roi = roi_mask.reshape(ROWS, N)
    wt = W.T
    b2 = b.reshape(H, 1)

    RB = 64
    out1s, masks = [], []
    offs = [sum(_SLICES[:s]) for s in range(len(_SLICES))]
    for s, sr in enumerate(_SLICES):
        off = offs[s] // RB
        out1_s, scores_s = pl.pallas_call(
            _stage1_body,
            grid=(sr // RB,),
            in_specs=[
                pl.BlockSpec((RB, C, N), lambda i, o=off: (i + o, 0, 0)),
                pl.BlockSpec((RB, C, N), lambda i, o=off: (i + o, 0, 0)),
                pl.BlockSpec((RB, 2, N), lambda i, o=off: (i + o, 0, 0)),
                pl.BlockSpec((RB, 2, N), lambda i, o=off: (i + o, 0, 0)),
                pl.BlockSpec((RB, N), lambda i, o=off: (i + o, 0)),
                pl.BlockSpec((H, 2 * C + 4), lambda i: (0, 0)),
                pl.BlockSpec((H, 1), lambda i: (0, 0)),
            ],
            out_specs=[
                pl.BlockSpec((RB, H, N), lambda i: (i, 0, 0)),
                pl.BlockSpec((2 * RB, N // 2), lambda i: (i, 0)),
            ],
            out_shape=[
                jax.ShapeDtypeStruct((sr, H, N), jnp.bfloat16),
                jax.ShapeDtypeStruct((2 * sr, N // 2), jnp.float32),
            ],
        )(q, k, sq, sk, roi, wt, b2)
        out1s.append(out1_s)
        masks.append(
            _sc_topk_union(sr // _NW)(scores_s.reshape(-1)).reshape(_NW, N))

    fill = jnp.minimum(node_num, 1).astype(jnp.float32)
    nm = len(_SLICES) * _NW
    maskall = jnp.concatenate(masks, axis=0) * fill

    RB2 = 128
    out = None
    for s, sr in enumerate(_SLICES):
        in_specs = [
            pl.BlockSpec((RB2, H, N), lambda i: (i, 0, 0)),
            pl.BlockSpec((nm, N), lambda i: (0, 0)),
        ]
        args = [out1s[s], maskall]
        body = _stage3_body
        aliases = {}
        if out is not None:
            in_specs = [pl.BlockSpec(memory_space=pl.ANY)] + in_specs
            args = [out] + args
            body = _stage3_acc_body
            aliases = {0: 0}
        off2 = offs[s] // RB2
        out = pl.pallas_call(
            body,
            grid=(sr // RB2,),
            in_specs=in_specs,
            out_specs=pl.BlockSpec((RB2, H, N),
                                   lambda i, o=off2: (i + o, 0, 0)),
            out_shape=jax.ShapeDtypeStruct((ROWS, H, N), jnp.float32),
            input_output_aliases=aliases,
        )(*args)
    return out.reshape(B, N, H, N).transpose(0, 1, 3, 2)


# speedup vs baseline: 1.0029x; 1.0029x over previous
import functools

import jax
import jax.numpy as jnp
from jax.experimental import pallas as pl
from jax.experimental.pallas import tpu as pltpu
from jax.experimental.pallas import tpu_sc as plsc

H = 8
N = 256
C = 64
ROWS = 8 * N
K = 32

_NW = 32
_SLICES = (512, 640, 640, 256)
_GRP = N // 16


def _stage1_body(q_ref, k_ref, sq_ref, sk_ref, roi_ref, w_ref, b_ref,
                 out1_ref, sc_ref):
    q = q_ref[...]
    k = k_ref[...]
    rb = q.shape[0]
    dot = jnp.sum(q * k, axis=1)
    qn = jnp.maximum(jnp.sqrt(jnp.sum(q * q, axis=1)), 1e-8)
    kn = jnp.maximum(jnp.sqrt(jnp.sum(k * k, axis=1)), 1e-8)
    scores = jnp.maximum(dot / (qn * kn), 0.0)
    sc_ref[...] = scores.reshape(2 * rb, N // 2)

    w = w_ref[...]
    wq = jnp.broadcast_to(w[None, :, 0:C], (rb, H, C))
    wk = jnp.broadcast_to(w[None, :, C:2 * C], (rb, H, C))
    z = jax.lax.dot_general(wq, q, (((2,), (1,)), ((0,), (0,))),
                            preferred_element_type=jnp.float32)
    z = z + jax.lax.dot_general(wk, k, (((2,), (1,)), ((0,), (0,))),
                                preferred_element_type=jnp.float32)
    sq = sq_ref[...]
    sk = sk_ref[...]
    z = (z
         + w[None, :, 2 * C:2 * C + 1] * sq[:, 0:1, :]
         + w[None, :, 2 * C + 1:2 * C + 2] * sq[:, 1:2, :]
         + w[None, :, 2 * C + 2:2 * C + 3] * sk[:, 0:1, :]
         + w[None, :, 2 * C + 3:2 * C + 4] * sk[:, 1:2, :])
    z = z + b_ref[...][None, :, :]
    attn1 = 1.0 / (1.0 + jnp.exp(-z))
    out1_ref[...] = (attn1 * roi_ref[...][:, None, :]).astype(jnp.bfloat16)


def _dyn_gather(x, idx):
    return jax.lax.gather(
        x, idx[:, None],
        dimension_numbers=jax.lax.GatherDimensionNumbers(
            offset_dims=(), collapsed_slice_dims=(0,), start_index_map=(0,)),
        slice_sizes=(1,), mode=jax.lax.GatherScatterMode.PROMISE_IN_BOUNDS)


def _vsum16(v):
    i = jax.lax.iota(jnp.int32, 16)
    for s in (8, 4, 2, 1):
        v = v + _dyn_gather(v, i ^ s)
    return v


def _prefix16(v):
    i = jax.lax.iota(jnp.int32, 16)
    for s in (1, 2, 4, 8):
        sh = _dyn_gather(v, jnp.maximum(i - s, 0))
        v = v + jnp.where(i >= s, sh, 0)
    return v


def _sc_topk_union_body(rpw, scores_hbm, out_hbm, rows_v, buf_v):
    wid = jax.lax.axis_index("c") * 16 + jax.lax.axis_index("s")
    pltpu.sync_copy(scores_hbm.at[pl.ds(wid * (rpw * N), rpw * N)], rows_v)
    lane15 = jnp.full((16,), 15, jnp.int32)

    def row_body(r, acc):
        start = r * N
        bs = [jax.lax.bitcast_convert_type(rows_v[pl.ds(start + g * 16, 16)],
                                           jnp.int32)
              for g in range(_GRP)]

        def bit_body(i, t):
            cand = t | jax.lax.shift_left(jnp.int32(1), jnp.int32(30) - i)
            cnt = jnp.zeros((16,), jnp.int32)
            for bvec in bs:
                cnt = cnt + jnp.where(bvec >= cand, 1, 0)
            return jnp.where(_vsum16(cnt) >= K, cand, t)

        t = jax.lax.fori_loop(0, 31, bit_body, jnp.zeros((16,), jnp.int32))

        cg = jnp.zeros((16,), jnp.int32)
        for bvec in bs:
            cg = cg + jnp.where(bvec > t, 1, 0)
        rem = K - _vsum16(cg)
        new_acc = []
        c = jnp.zeros((16,), jnp.int32)
        for g in range(_GRP):
            eq = bs[g] == t
            pref = _prefix16(jnp.where(eq, 1, 0)) + c
            keep = (bs[g] > t) | (eq & (pref <= rem))
            new_acc.append(jnp.maximum(acc[g], jnp.where(keep, 1.0, 0.0)))
            c = _dyn_gather(pref, lane15)
        return tuple(new_acc)

    acc0 = tuple(jnp.zeros((16,), jnp.float32) for _ in range(_GRP))
    acc = jax.lax.fori_loop(0, rpw, row_body, acc0)
    for g in range(_GRP):
        buf_v[pl.ds(g * 16, 16)] = acc[g]
    pltpu.sync_copy(buf_v, out_hbm.at[pl.ds(wid * N, N)])


@functools.cache
def _sc_topk_union(rpw):
    return pl.kernel(
        functools.partial(_sc_topk_union_body, rpw),
        out_type=jax.ShapeDtypeStruct((_NW * N,), jnp.float32),
        mesh=plsc.VectorSubcoreMesh(core_axis_name="c", subcore_axis_name="s"),
        scratch_types=[
            pltpu.VMEM((rpw * N,), jnp.float32),
            pltpu.VMEM((N,), jnp.float32),
        ],
    )


def _stage3_body(out1_ref, m_ref, o_ref):
    colmask = jnp.max(m_ref[...], axis=0)
    o_ref[...] = out1_ref[...].astype(jnp.float32) * colmask[None, None, :]


def _stage3_acc_body(prev_ref, out1_ref, m_ref, o_ref):
    del prev_ref
    _stage3_body(out1_ref, m_ref, o_ref)


def kernel(query, key_t, s_query, s_key, roi_mask, W, b, node_num):
    B = query.shape[0]
    q = query.transpose(0, 1, 3, 2).reshape(ROWS, C, N)
    k = key_t.transpose(0, 1, 3, 2).reshape(ROWS, C, N)
    sq = s_query.transpose(0, 1, 3, 2).reshape(ROWS, 2, N)
    sk = s_key.transpose(0, 1, 3, 2).reshape(ROWS, 2, N)
    roi = roi_mask.reshape(ROWS, N)
    wt = W.T
    b2 = b.reshape(H, 1)

    RB = 64
    out1s, masks = [], []
    offs = [sum(_SLICES[:s]) for s in range(len(_SLICES))]
    for s, sr in enumerate(_SLICES):
        off = offs[s] // RB
        out1_s, scores_s = pl.pallas_call(
            _stage1_body,
            grid=(sr // RB,),
            in_specs=[
                pl.BlockSpec((RB, C, N), lambda i, o=off: (i + o, 0, 0)),
                pl.BlockSpec((RB, C, N), lambda i, o=off: (i + o, 0, 0)),
                pl.BlockSpec((RB, 2, N), lambda i, o=off: (i + o, 0, 0)),
                pl.BlockSpec((RB, 2, N), lambda i, o=off: (i + o, 0, 0)),
                pl.BlockSpec((RB, N), lambda i, o=off: (i + o, 0)),
                pl.BlockSpec((H, 2 * C + 4), lambda i: (0, 0)),
                pl.BlockSpec((H, 1), lambda i: (0, 0)),
            ],
            out_specs=[
                pl.BlockSpec((RB, H, N), lambda i: (i, 0, 0)),
                pl.BlockSpec((2 * RB, N // 2), lambda i: (i, 0)),
            ],
            out_shape=[
                jax.ShapeDtypeStruct((sr, H, N), jnp.bfloat16),
                jax.ShapeDtypeStruct((2 * sr, N // 2), jnp.float32),
            ],
        )(q, k, sq, sk, roi, wt, b2)
        out1s.append(out1_s)
        masks.append(
            _sc_topk_union(sr // _NW)(scores_s.reshape(-1)).reshape(_NW, N))

    fill = jnp.minimum(node_num, 1).astype(jnp.float32)
    nm = len(_SLICES) * _NW
    maskall = jnp.concatenate(masks, axis=0) * fill

    RB2 = 128
    out = None
    for s, sr in enumerate(_SLICES):
        in_specs = [
            pl.BlockSpec((RB2, H, N), lambda i: (i, 0, 0)),
            pl.BlockSpec((nm, N), lambda i: (0, 0)),
        ]
        args = [out1s[s], maskall]
        body = _stage3_body
        aliases = {}
        if out is not None:
            in_specs = [pl.BlockSpec(memory_space=pl.ANY)] + in_specs
            args = [out] + args
            body = _stage3_acc_body
            aliases = {0: 0}
        off2 = offs[s] // RB2
        out = pl.pallas_call(
            body,
            grid=(sr // RB2,),
            in_specs=in_specs,
            out_specs=pl.BlockSpec((RB2, H, N),
                                   lambda i, o=off2: (i + o, 0, 0)),
            out_shape=jax.ShapeDtypeStruct((ROWS, H, N), jnp.float32),
            input_output_aliases=aliases,
        )(*args)
    return out.reshape(B, N, H, N).transpose(0, 1, 3, 2)
